# Initial kernel scaffold; baseline (speedup 1.0000x reference)
import jax
import jax.numpy as jnp
from jax.experimental import pallas as pl


def kernel(premise, hypothesis, emb_table, fc_w, fc_b):
    raise NotImplementedError("write your pallas kernel here")


# speedup vs baseline: 1.5102x; 1.5102x over previous
import functools

import jax
import jax.numpy as jnp
from jax import lax
from jax.experimental import pallas as pl
from jax.experimental.pallas import tpu as pltpu
from jax.experimental.pallas import tpu_sc as plsc

L = 16
NC, NS = 2, 16
NW = NC * NS

ROWS_PER_CHUNK = 2


def _sc_pool_kernel(n_rows, seq, d, rows_per_w):
    g_per_w = rows_per_w // ROWS_PER_CHUNK
    idx_per_chunk = ROWS_PER_CHUNK * seq
    dchunks = d // L
    inv = 1.0 / seq

    def body(idx_hbm, tab_hbm, out_hbm, idx_v, buf0, buf1, pooled_v, sem0, sem1):
        cid = lax.axis_index("c")
        sid = lax.axis_index("s")
        wid = sid * NC + cid
        gbase = wid * g_per_w

        pltpu.sync_copy(idx_hbm.at[pl.ds(gbase, g_per_w)], idx_v)

        pltpu.async_copy(tab_hbm.at[idx_v.at[0]], buf0, sem0)
        pltpu.async_copy(tab_hbm.at[idx_v.at[1]], buf1, sem1)

        def reduce_chunk(buf, g):
            def s_body(s, accs):
                out = []
                for r in range(ROWS_PER_CHUNK):
                    for c in range(dchunks):
                        out.append(accs[r * dchunks + c]
                                   + buf[r * seq + s, pl.ds(c * L, L)])
                return tuple(out)
            zeros = tuple(jnp.zeros((L,), jnp.float32)
                          for _ in range(ROWS_PER_CHUNK * dchunks))
            accs = lax.fori_loop(0, seq, s_body, zeros)
            for r in range(ROWS_PER_CHUNK):
                for c in range(dchunks):
                    pooled_v[ROWS_PER_CHUNK * g + r, pl.ds(c * L, L)] = (
                        accs[r * dchunks + c] * inv)

        def step(i, carry):
            for slot, (buf, sem) in enumerate(((buf0, sem0), (buf1, sem1))):
                g = 2 * i + slot
                pltpu.make_async_copy(tab_hbm.at[idx_v.at[g]], buf, sem).wait()
                reduce_chunk(buf, g)

                @pl.when(g + 2 < g_per_w)
                def _():
                    pltpu.async_copy(tab_hbm.at[idx_v.at[g + 2]], buf, sem)
            return carry

        lax.fori_loop(0, g_per_w // 2, step, 0)

        pltpu.sync_copy(pooled_v, out_hbm.at[pl.ds(wid * rows_per_w, rows_per_w)])

    return pl.kernel(
        body,
        out_type=jax.ShapeDtypeStruct((n_rows, d), jnp.float32),
        mesh=plsc.VectorSubcoreMesh(core_axis_name="c", subcore_axis_name="s",
                                    num_cores=NC, num_subcores=NS),
        scratch_types=[
            pltpu.VMEM((g_per_w, idx_per_chunk), jnp.int32),
            pltpu.VMEM((idx_per_chunk, d), jnp.float32),
            pltpu.VMEM((idx_per_chunk, d), jnp.float32),
            pltpu.VMEM((rows_per_w, d), jnp.float32),
            pltpu.SemaphoreType.DMA,
            pltpu.SemaphoreType.DMA,
        ],
        compiler_params=pltpu.CompilerParams(use_tc_tiling_on_sc=False),
    )


def _linear_body(xp_ref, xh_ref, w_ref, b_ref, o_ref):
    d = xp_ref.shape[1]
    o_ref[...] = (
        jnp.dot(xp_ref[...], w_ref[0:d, :], preferred_element_type=jnp.float32)
        + jnp.dot(xh_ref[...], w_ref[d:2 * d, :], preferred_element_type=jnp.float32)
        + b_ref[...]
    )


def kernel(premise, hypothesis, emb_table, fc_w, fc_b):
    b, seq = premise.shape
    _, d = emb_table.shape
    odim = fc_w.shape[0]
    n_rows = 2 * b
    rows_per_w = n_rows // NW

    idx = jnp.concatenate(
        [premise.astype(jnp.int32), hypothesis.astype(jnp.int32)], axis=0)
    idx2 = idx.reshape(n_rows // ROWS_PER_CHUNK, ROWS_PER_CHUNK * seq)

    pooled = _sc_pool_kernel(n_rows, seq, d, rows_per_w)(idx2, emb_table)

    wpad = jnp.zeros((2 * d, 128), jnp.float32).at[:, :odim].set(fc_w.T)
    bpad = jnp.zeros((1, 128), jnp.float32).at[0, :odim].set(fc_b)

    bm = 512
    out_pad = pl.pallas_call(
        _linear_body,
        grid=(b // bm,),
        in_specs=[
            pl.BlockSpec((bm, d), lambda i: (i, 0)),
            pl.BlockSpec((bm, d), lambda i: (i + b // bm, 0)),
            pl.BlockSpec((2 * d, 128), lambda i: (0, 0)),
            pl.BlockSpec((1, 128), lambda i: (0, 0)),
        ],
        out_specs=pl.BlockSpec((bm, 128), lambda i: (i, 0)),
        out_shape=jax.ShapeDtypeStruct((b, 128), jnp.float32),
    )(pooled, pooled, wpad, bpad)

    return out_pad[:, :odim]


# speedup vs baseline: 1.5418x; 1.0209x over previous
import jax
import jax.numpy as jnp
from jax import lax
from jax.experimental import pallas as pl
from jax.experimental.pallas import tpu as pltpu
from jax.experimental.pallas import tpu_sc as plsc

L = 16
NC, NS = 2, 16
NW = NC * NS
NBUF = 4


def _sc_pool_kernel(b, seq, d):
    rows_per_w = b // NW
    dchunks = d // L
    inv = 1.0 / seq

    def body(p_hbm, h_hbm, tab_hbm, outp_hbm, outh_hbm,
             idxp_v, idxh_v, b0, b1, b2, b3, pool_v, s0, s1, s2, s3):
        cid = lax.axis_index("c")
        sid = lax.axis_index("s")
        wid = sid * NC + cid
        base = wid * rows_per_w

        bufs = (b0, b1, b2, b3)
        sems = (s0, s1, s2, s3)

        pltpu.sync_copy(p_hbm.at[pl.ds(base, rows_per_w)], idxp_v)
        pltpu.sync_copy(h_hbm.at[pl.ds(base, rows_per_w)], idxh_v)

        zeros = tuple(jnp.zeros((L,), jnp.float32) for _ in range(dchunks))

        for idx_v, out_hbm in ((idxp_v, outp_hbm), (idxh_v, outh_hbm)):
            for k in range(NBUF):
                pltpu.async_copy(tab_hbm.at[idx_v.at[k]], bufs[k], sems[k])

            def step(i, carry):
                for k in range(NBUF):
                    g = NBUF * i + k
                    pltpu.make_async_copy(
                        tab_hbm.at[idx_v.at[g]], bufs[k], sems[k]).wait()

                    def s_body(s, accs, _buf=bufs[k]):
                        return tuple(accs[c] + _buf[s, pl.ds(c * L, L)]
                                     for c in range(dchunks))
                    accs = lax.fori_loop(0, seq, s_body, zeros)
                    for c in range(dchunks):
                        pool_v[g, pl.ds(c * L, L)] = accs[c] * inv

                    @pl.when(g + NBUF < rows_per_w)
                    def _():
                        pltpu.async_copy(
                            tab_hbm.at[idx_v.at[g + NBUF]], bufs[k], sems[k])
                return carry

            lax.fori_loop(0, rows_per_w // NBUF, step, 0)
            pltpu.sync_copy(pool_v, out_hbm.at[pl.ds(base, rows_per_w)])

    return pl.kernel(
        body,
        out_type=[jax.ShapeDtypeStruct((b, d), jnp.float32),
                  jax.ShapeDtypeStruct((b, d), jnp.float32)],
        mesh=plsc.VectorSubcoreMesh(core_axis_name="c", subcore_axis_name="s",
                                    num_cores=NC, num_subcores=NS),
        scratch_types=(
            [pltpu.VMEM((rows_per_w, seq), jnp.int32)] * 2
            + [pltpu.VMEM((seq, d), jnp.float32)] * NBUF
            + [pltpu.VMEM((rows_per_w, d), jnp.float32)]
            + [pltpu.SemaphoreType.DMA] * NBUF
        ),
        compiler_params=pltpu.CompilerParams(use_tc_tiling_on_sc=False),
    )


def _linear_body(xp_ref, xh_ref, w_ref, b_ref, o_ref):
    d = xp_ref.shape[1]
    o_ref[...] = (
        jnp.dot(xp_ref[...], w_ref[0:d, :], preferred_element_type=jnp.float32)
        + jnp.dot(xh_ref[...], w_ref[d:2 * d, :], preferred_element_type=jnp.float32)
        + b_ref[...]
    )


def kernel(premise, hypothesis, emb_table, fc_w, fc_b):
    b, seq = premise.shape
    _, d = emb_table.shape
    odim = fc_w.shape[0]

    pooled_p, pooled_h = _sc_pool_kernel(b, seq, d)(
        premise.astype(jnp.int32), hypothesis.astype(jnp.int32), emb_table)

    wpad = jnp.zeros((2 * d, 128), jnp.float32).at[:, :odim].set(fc_w.T)
    bpad = jnp.zeros((1, 128), jnp.float32).at[0, :odim].set(fc_b)

    bm = 512
    out_pad = pl.pallas_call(
        _linear_body,
        grid=(b // bm,),
        in_specs=[
            pl.BlockSpec((bm, d), lambda i: (i, 0)),
            pl.BlockSpec((bm, d), lambda i: (i, 0)),
            pl.BlockSpec((2 * d, 128), lambda i: (0, 0)),
            pl.BlockSpec((1, 128), lambda i: (0, 0)),
        ],
        out_specs=pl.BlockSpec((bm, 128), lambda i: (i, 0)),
        out_shape=jax.ShapeDtypeStruct((b, 128), jnp.float32),
    )(pooled_p, pooled_h, wpad, bpad)

    return out_pad[:, :odim]


# speedup vs baseline: 1.6441x; 1.0663x over previous
import jax
import jax.numpy as jnp
from jax import lax
from jax.experimental import pallas as pl
from jax.experimental.pallas import tpu as pltpu
from jax.experimental.pallas import tpu_sc as plsc

L = 16
NC, NS = 2, 16
NW = NC * NS
NBUF = 4


def _sc_pool_kernel(b, seq, d, dpad):
    rows_per_w = b // NW
    dchunks = d // L
    inv = 1.0 / seq

    def body(p_hbm, h_hbm, tab_hbm, outp_hbm, outh_hbm,
             idxp_v, idxh_v, b0, b1, b2, b3, pool_v, s0, s1, s2, s3):
        cid = lax.axis_index("c")
        sid = lax.axis_index("s")
        wid = sid * NC + cid
        base = wid * rows_per_w

        bufs = (b0, b1, b2, b3)
        sems = (s0, s1, s2, s3)

        pltpu.sync_copy(p_hbm.at[pl.ds(base, rows_per_w)], idxp_v)
        pltpu.sync_copy(h_hbm.at[pl.ds(base, rows_per_w)], idxh_v)

        zeros = tuple(jnp.zeros((L,), jnp.float32) for _ in range(dchunks))

        for idx_v, out_hbm in ((idxp_v, outp_hbm), (idxh_v, outh_hbm)):
            for k in range(NBUF):
                pltpu.async_copy(tab_hbm.at[idx_v.at[k]], bufs[k], sems[k])

            def step(i, carry):
                for k in range(NBUF):
                    g = NBUF * i + k
                    pltpu.make_async_copy(
                        tab_hbm.at[idx_v.at[g]], bufs[k], sems[k]).wait()

                    def s_body(s, accs, _buf=bufs[k]):
                        return tuple(accs[c] + _buf[s, pl.ds(c * L, L)]
                                     for c in range(dchunks))
                    accs = lax.fori_loop(0, seq, s_body, zeros)
                    for c in range(dchunks):
                        pool_v[g, pl.ds(c * L, L)] = accs[c] * inv

                    @pl.when(g + NBUF < rows_per_w)
                    def _():
                        pltpu.async_copy(
                            tab_hbm.at[idx_v.at[g + NBUF]], bufs[k], sems[k])
                return carry

            lax.fori_loop(0, rows_per_w // NBUF, step, 0)
            pltpu.sync_copy(pool_v, out_hbm.at[pl.ds(base, rows_per_w)])

    return pl.kernel(
        body,
        out_type=[jax.ShapeDtypeStruct((b, d), jnp.float32),
                  jax.ShapeDtypeStruct((b, d), jnp.float32)],
        mesh=plsc.VectorSubcoreMesh(core_axis_name="c", subcore_axis_name="s",
                                    num_cores=NC, num_subcores=NS),
        scratch_types=(
            [pltpu.VMEM((rows_per_w, seq), jnp.int32)] * 2
            + [pltpu.VMEM((seq, dpad), jnp.float32)] * NBUF
            + [pltpu.VMEM((rows_per_w, d), jnp.float32)]
            + [pltpu.SemaphoreType.DMA] * NBUF
        ),
        compiler_params=pltpu.CompilerParams(use_tc_tiling_on_sc=True),
    )


def _linear_body(xp_ref, xh_ref, w_ref, b_ref, o_ref):
    d = xp_ref.shape[1]
    o_ref[...] = (
        jnp.dot(xp_ref[...], w_ref[0:d, :], preferred_element_type=jnp.float32)
        + jnp.dot(xh_ref[...], w_ref[d:2 * d, :], preferred_element_type=jnp.float32)
        + b_ref[...]
    )


def kernel(premise, hypothesis, emb_table, fc_w, fc_b):
    b, seq = premise.shape
    _, d = emb_table.shape
    odim = fc_w.shape[0]

    dpad = 128
    tab128 = jnp.pad(emb_table, ((0, 0), (0, dpad - d)))

    pooled_p, pooled_h = _sc_pool_kernel(b, seq, d, dpad)(
        premise.astype(jnp.int32), hypothesis.astype(jnp.int32), tab128)

    wpad = jnp.zeros((2 * d, 128), jnp.float32).at[:, :odim].set(fc_w.T)
    bpad = jnp.zeros((1, 128), jnp.float32).at[0, :odim].set(fc_b)

    bm = 512
    out_pad = pl.pallas_call(
        _linear_body,
        grid=(b // bm,),
        in_specs=[
            pl.BlockSpec((bm, d), lambda i: (i, 0)),
            pl.BlockSpec((bm, d), lambda i: (i, 0)),
            pl.BlockSpec((2 * d, 128), lambda i: (0, 0)),
            pl.BlockSpec((1, 128), lambda i: (0, 0)),
        ],
        out_specs=pl.BlockSpec((bm, 128), lambda i: (i, 0)),
        out_shape=jax.ShapeDtypeStruct((b, 128), jnp.float32),
    )(pooled_p, pooled_h, wpad, bpad)

    return out_pad[:, :odim]
